# Initial kernel scaffold; baseline (speedup 1.0000x reference)
import jax
import jax.numpy as jnp
from jax.experimental import pallas as pl


def kernel(x, edge_index, W1, b1, g1, be1, W2, b2, g2, be2):
    raise NotImplementedError("write your pallas kernel here")


# speedup vs baseline: 14.0808x; 14.0808x over previous
import functools

import jax
import jax.numpy as jnp
from jax import lax
from jax.experimental import pallas as pl
from jax.experimental.pallas import tpu as pltpu
from jax.experimental.pallas import tpu_sc as plsc

N = 10000
D = 128
E = 320000
EPS = 1e-5

NC = 2
NS = 16
NW = NC * NS
CHUNK = 128
CPW = -(-E // (NW * CHUNK))
EPW = CPW * CHUNK
E_PAD = NW * EPW
RPT = 640
N_ACC = NS * RPT

_mesh = plsc.VectorSubcoreMesh(core_axis_name="c", subcore_axis_name="s")


@functools.partial(
    pl.kernel,
    mesh=_mesh,
    out_type=jax.ShapeDtypeStruct((NC * N_ACC,), jnp.float32),
    scratch_types=[
        pltpu.VMEM((CPW, CHUNK), jnp.int32),
        pltpu.VMEM((CHUNK,), jnp.float32),
        pltpu.VMEM_SHARED((N_ACC,), jnp.float32),
    ],
)
def _deg_kernel(dst_hbm, zvec_hbm, out_hbm, dst_v, ones_v, deg_sh):
    cid = lax.axis_index("c")
    sid = lax.axis_index("s")
    wid = cid * NS + sid
    pltpu.sync_copy(zvec_hbm, deg_sh.at[pl.ds(sid * RPT, RPT)])
    pltpu.sync_copy(dst_hbm.at[wid], dst_v)

    one16 = jnp.full((16,), 1.0, jnp.float32)

    def obody(i, carry):
        ones_v[pl.ds(i * 16, 16)] = one16
        return carry

    lax.fori_loop(0, CHUNK // 16, obody, 0)
    plsc.subcore_barrier()

    def ebody(j, carry):
        pltpu.sync_copy(ones_v, deg_sh.at[dst_v.at[j]], add=True)
        return carry

    lax.fori_loop(0, CPW, ebody, 0)
    plsc.subcore_barrier()
    pltpu.sync_copy(deg_sh.at[pl.ds(sid * RPT, RPT)],
                    out_hbm.at[pl.ds(cid * N_ACC + sid * RPT, RPT)])


@functools.partial(
    pl.kernel,
    mesh=_mesh,
    out_type=jax.ShapeDtypeStruct((NC, N_ACC, D), jnp.float32),
    scratch_types=[
        pltpu.VMEM((CPW, CHUNK), jnp.int32),
        pltpu.VMEM((CPW, CHUNK), jnp.int32),
        pltpu.VMEM((CHUNK, D), jnp.float32),
        pltpu.VMEM_SHARED((N_ACC, D), jnp.float32),
        pltpu.SemaphoreType.DMA,
    ],
)
def _agg_kernel(src_hbm, dst_hbm, hs_hbm, zrows_hbm, out_hbm,
                src_v, dst_v, rows_v, acc_sh, sem):
    cid = lax.axis_index("c")
    sid = lax.axis_index("s")
    wid = cid * NS + sid
    pltpu.sync_copy(zrows_hbm, acc_sh.at[pl.ds(sid * RPT, RPT)])
    pltpu.sync_copy(src_hbm.at[wid], src_v)
    pltpu.sync_copy(dst_hbm.at[wid], dst_v)
    plsc.subcore_barrier()

    def ebody(j, carry):
        pltpu.async_copy(hs_hbm.at[src_v.at[j]], rows_v, sem).wait()
        pltpu.sync_copy(rows_v, acc_sh.at[dst_v.at[j]], add=True)
        return carry

    lax.fori_loop(0, CPW, ebody, 0)
    plsc.subcore_barrier()
    pltpu.sync_copy(acc_sh.at[pl.ds(sid * RPT, RPT)],
                    out_hbm.at[cid, pl.ds(sid * RPT, RPT)])


def _tc1_body(degT_ref, x_ref, W1_ref, b1_ref, dinv_ref, hs_ref):
    deg = jnp.sum(degT_ref[...], axis=1, keepdims=True) + 1.0
    dinv = lax.rsqrt(deg[:N])
    h = jnp.dot(x_ref[...], W1_ref[...],
                preferred_element_type=jnp.float32) + b1_ref[...]
    dinv_ref[...] = dinv
    hs_ref[...] = h * dinv


_tc1_call = pl.pallas_call(
    _tc1_body,
    out_shape=[
        jax.ShapeDtypeStruct((N, 1), jnp.float32),
        jax.ShapeDtypeStruct((N, D), jnp.float32),
    ],
)


def _tc2_body(agg_ref, hs1_ref, dinv_ref, g_ref, be_ref, W2_ref, b2_ref,
              out_ref):
    t = (agg_ref[0, :N, :] + agg_ref[1, :N, :] + hs1_ref[...]) * dinv_ref[...]
    mean = jnp.mean(t, axis=0, keepdims=True)
    var = jnp.mean(t * t, axis=0, keepdims=True) - mean * mean
    y = g_ref[...] * (t - mean) * lax.rsqrt(var + EPS) + be_ref[...]
    y = jnp.maximum(y, 0.0)
    h2 = jnp.dot(y, W2_ref[...],
                 preferred_element_type=jnp.float32) + b2_ref[...]
    out_ref[...] = h2 * dinv_ref[...]


_tc2_call = pl.pallas_call(
    _tc2_body,
    out_shape=jax.ShapeDtypeStruct((N, D), jnp.float32),
)


def _tc3_body(agg_ref, hs2_ref, dinv_ref, g_ref, be_ref, out_ref):
    t = (agg_ref[0, :N, :] + agg_ref[1, :N, :] + hs2_ref[...]) * dinv_ref[...]
    mean = jnp.mean(t, axis=0, keepdims=True)
    var = jnp.mean(t * t, axis=0, keepdims=True) - mean * mean
    out_ref[...] = g_ref[...] * (t - mean) * lax.rsqrt(var + EPS) + be_ref[...]


_tc3_call = pl.pallas_call(
    _tc3_body,
    out_shape=jax.ShapeDtypeStruct((N, D), jnp.float32),
)


def kernel(x, edge_index, W1, b1, g1, be1, W2, b2, g2, be2):
    src = edge_index[0]
    dst = edge_index[1]
    pad = E_PAD - E
    src_p = jnp.concatenate([src, jnp.zeros((pad,), jnp.int32)])
    dst_p = jnp.concatenate([dst, jnp.full((pad,), N, jnp.int32)])
    src3 = src_p.reshape(NW, CPW, CHUNK)
    dst3 = dst_p.reshape(NW, CPW, CHUNK)
    zrows = jnp.zeros((RPT, D), jnp.float32)
    zvec = jnp.zeros((RPT,), jnp.float32)

    deg_parts = _deg_kernel(dst3, zvec)
    degT = deg_parts.reshape(NC, N_ACC).T
    dinv, hs1 = _tc1_call(degT, x, W1, b1)
    agg1 = _agg_kernel(src3, dst3, hs1, zrows)
    hs2 = _tc2_call(agg1, hs1, dinv, g1, be1, W2, b2)
    agg2 = _agg_kernel(src3, dst3, hs2, zrows)
    return _tc3_call(agg2, hs2, dinv, g2, be2)
